# baseline (speedup 1.0000x reference)
import jax
import jax.numpy as jnp
from jax import lax
from jax.experimental import pallas as pl
from jax.experimental.pallas import tpu as pltpu

K = 32
KK = K * K
XMIN, XMAX = -15.0, 15.0
YMIN, YMAX = -15.0, 15.0
EPS = 1e-06


def _splat_body(coord_ref, feat_ref, out_ref, w_ref, s_scr):
    j = pl.program_id(1)

    @pl.when(j == 0)
    def _build_and_weights():
        cxy = coord_ref[0]
        P = cxy.shape[0]
        cx = cxy[:, 0:1]
        cy = cxy[:, 1:2]
        gx = (cx - XMIN) / max(XMAX - XMIN, 1e-06) * (K - 1)
        gy = (cy - YMIN) / max(YMAX - YMIN, 1e-06) * (K - 1)
        x0 = jnp.floor(gx)
        y0 = jnp.floor(gy)
        x1 = x0 + 1.0
        y1 = y0 + 1.0
        wx1 = gx - x0
        wy1 = gy - y0
        wx0 = 1.0 - wx1
        wy0 = 1.0 - wy1
        cells = lax.broadcasted_iota(jnp.int32, (P, KK), 1)
        lanes32 = lax.broadcasted_iota(jnp.int32, (P, K), 1)
        S = jnp.zeros((P, KK), dtype=jnp.float32)
        w2d = jnp.zeros((K, K), dtype=jnp.float32)
        for nx, ny, w in ((x0, y0, wx0 * wy0), (x1, y0, wx1 * wy0),
                          (x0, y1, wx0 * wy1), (x1, y1, wx1 * wy1)):
            valid = ((nx >= 0) & (nx < K) & (ny >= 0) & (ny < K))
            ix = jnp.clip(nx, 0, K - 1).astype(jnp.int32)
            iy = jnp.clip(ny, 0, K - 1).astype(jnp.int32)
            idx = iy * K + ix
            wv = jnp.where(valid, w, 0.0)
            S = S + jnp.where(idx == cells, wv, 0.0)
            yhot_w = jnp.where(iy == lanes32, wv, 0.0)
            xhot = jnp.where(ix == lanes32, 1.0, 0.0)
            w2d = w2d + lax.dot_general(
                yhot_w, xhot, (((0,), (0,)), ((), ())),
                preferred_element_type=jnp.float32,
                precision=lax.Precision.HIGHEST)
        colsum = jnp.sum(S, axis=0)
        s_scr[...] = S * (1.0 / jnp.clip(colsum, EPS, None))[None, :]
        for ti in range(w_ref.shape[1]):
            w_ref[0, ti, 0, :, :] = w2d

    out_ref[0] = jnp.dot(feat_ref[0], s_scr[...],
                         preferred_element_type=jnp.float32,
                         precision=lax.Precision.DEFAULT)


def kernel(patch_features, coord_map):
    b, t, d, hp, wp = patch_features.shape
    P = hp * wp
    TD = t * d
    TDB = 512
    feats = patch_features.reshape(b, TD, P)
    coords = coord_map.reshape(b, P, 2)

    grid = (b, TD // TDB)
    world, weights = pl.pallas_call(
        _splat_body,
        grid=grid,
        in_specs=[
            pl.BlockSpec((1, P, 2), lambda i, j: (i, 0, 0)),
            pl.BlockSpec((1, TDB, P), lambda i, j: (i, j, 0)),
        ],
        out_specs=[
            pl.BlockSpec((1, TDB, KK), lambda i, j: (i, j, 0)),
            pl.BlockSpec((1, t, 1, K, K), lambda i, j: (i, 0, 0, 0, 0)),
        ],
        out_shape=[
            jax.ShapeDtypeStruct((b, TD, KK), jnp.float32),
            jax.ShapeDtypeStruct((b, t, 1, K, K), jnp.float32),
        ],
        scratch_shapes=[pltpu.VMEM((P, KK), jnp.float32)],
    )(coords, feats)
    world = world.reshape(b, t, d, K, K)
    return (world, weights)


# speedup vs baseline: 4.6793x; 4.6793x over previous
import jax
import jax.numpy as jnp
from jax import lax
from jax.experimental import pallas as pl
from jax.experimental.pallas import tpu as pltpu

K = 32
KK = K * K
XMIN, XMAX = -15.0, 15.0
YMIN, YMAX = -15.0, 15.0
EPS = 1e-06


def _splat_body(coord_ref, feat_ref, out_ref, w_ref, st_scr):
    i = pl.program_id(1)

    @pl.when(i == 0)
    def _build_and_weights():
        cx = coord_ref[0, 0, :]
        cy = coord_ref[0, 1, :]
        P = cx.shape[0]
        gx = (cx - XMIN) / max(XMAX - XMIN, 1e-06) * (K - 1)
        gy = (cy - YMIN) / max(YMAX - YMIN, 1e-06) * (K - 1)
        x0 = jnp.floor(gx)
        y0 = jnp.floor(gy)
        x1 = x0 + 1.0
        y1 = y0 + 1.0
        wx1 = gx - x0
        wy1 = gy - y0
        wx0 = 1.0 - wx1
        wy0 = 1.0 - wy1
        cells = lax.broadcasted_iota(jnp.int32, (KK, P), 0)
        St = jnp.zeros((KK, P), dtype=jnp.float32)
        for nx, ny, w in ((x0, y0, wx0 * wy0), (x1, y0, wx1 * wy0),
                          (x0, y1, wx0 * wy1), (x1, y1, wx1 * wy1)):
            valid = ((nx >= 0) & (nx < K) & (ny >= 0) & (ny < K))
            idx = (jnp.clip(ny, 0, K - 1) * K + jnp.clip(nx, 0, K - 1)).astype(jnp.int32)
            wv = jnp.where(valid, w, 0.0)
            St = St + jnp.where(idx[None, :] == cells, wv[None, :], 0.0)
        norm = jnp.sum(St, axis=1, keepdims=True)
        st_scr[...] = St * (1.0 / jnp.clip(norm, EPS, None))
        crow = lax.broadcasted_iota(jnp.int32, (KK, K), 0) // K
        ccol = lax.broadcasted_iota(jnp.int32, (KK, K), 0) % K
        lane = lax.broadcasted_iota(jnp.int32, (KK, K), 1)
        sel_y = jnp.where(crow == lane, 1.0, 0.0)
        sel_xw = jnp.where(ccol == lane, norm, 0.0)
        w2d = lax.dot_general(sel_y, sel_xw, (((0,), (0,)), ((), ())),
                              preferred_element_type=jnp.float32,
                              precision=lax.Precision.HIGHEST)
        for ti in range(w_ref.shape[1]):
            w_ref[0, ti, 0, :, :] = w2d

    for ti in range(feat_ref.shape[1]):
        out_ref[0, ti] = jnp.dot(st_scr[...], feat_ref[0, ti],
                                 preferred_element_type=jnp.float32,
                                 precision=lax.Precision.DEFAULT)


def kernel(patch_features, coord_map):
    b, t, d, hp, wp = patch_features.shape
    P = hp * wp
    TB = 4
    feats = patch_features.transpose(0, 1, 3, 4, 2).reshape(b, t, P, d)
    coords = coord_map.reshape(b, P, 2).transpose(0, 2, 1)

    grid = (b, t // TB)
    out, weights = pl.pallas_call(
        _splat_body,
        grid=grid,
        in_specs=[
            pl.BlockSpec((1, 2, P), lambda i, j: (i, 0, 0)),
            pl.BlockSpec((1, TB, P, d), lambda i, j: (i, j, 0, 0)),
        ],
        out_specs=[
            pl.BlockSpec((1, TB, KK, d), lambda i, j: (i, j, 0, 0)),
            pl.BlockSpec((1, t, 1, K, K), lambda i, j: (i, 0, 0, 0, 0)),
        ],
        out_shape=[
            jax.ShapeDtypeStruct((b, t, KK, d), jnp.float32),
            jax.ShapeDtypeStruct((b, t, 1, K, K), jnp.float32),
        ],
        scratch_shapes=[pltpu.VMEM((KK, P), jnp.float32)],
    )(coords, feats)
    world = out.reshape(b, t, K, K, d).transpose(0, 1, 4, 2, 3)
    return (world, weights)


# speedup vs baseline: 5.6672x; 1.2111x over previous
import jax
import jax.numpy as jnp
from jax import lax
from jax.experimental import pallas as pl
from jax.experimental.pallas import tpu as pltpu

K = 32
KK = K * K
XMIN, XMAX = -15.0, 15.0
YMIN, YMAX = -15.0, 15.0
EPS = 1e-06


def _splat_body(coord_ref, feat_ref, out_ref, w_ref, st_scr):
    i = pl.program_id(1)

    @pl.when(i == 0)
    def _build_and_weights():
        cx = coord_ref[0, 0, :]
        cy = coord_ref[0, 1, :]
        P = cx.shape[0]
        gx = (cx - XMIN) / max(XMAX - XMIN, 1e-06) * (K - 1)
        gy = (cy - YMIN) / max(YMAX - YMIN, 1e-06) * (K - 1)
        x0 = jnp.floor(gx)
        y0 = jnp.floor(gy)
        x1 = x0 + 1.0
        y1 = y0 + 1.0
        wx1 = gx - x0
        wy1 = gy - y0
        wx0 = 1.0 - wx1
        wy0 = 1.0 - wy1
        cells = lax.broadcasted_iota(jnp.int32, (KK, P), 0)
        St = jnp.zeros((KK, P), dtype=jnp.float32)
        for nx, ny, w in ((x0, y0, wx0 * wy0), (x1, y0, wx1 * wy0),
                          (x0, y1, wx0 * wy1), (x1, y1, wx1 * wy1)):
            valid = ((nx >= 0) & (nx < K) & (ny >= 0) & (ny < K))
            idx = (jnp.clip(ny, 0, K - 1) * K + jnp.clip(nx, 0, K - 1)).astype(jnp.int32)
            wv = jnp.where(valid, w, 0.0)
            St = St + jnp.where(idx[None, :] == cells, wv[None, :], 0.0)
        norm = jnp.sum(St, axis=1, keepdims=True)
        st_scr[...] = St * (1.0 / jnp.clip(norm, EPS, None))
        crow = lax.broadcasted_iota(jnp.int32, (KK, K), 0) // K
        ccol = lax.broadcasted_iota(jnp.int32, (KK, K), 0) % K
        lane = lax.broadcasted_iota(jnp.int32, (KK, K), 1)
        sel_y = jnp.where(crow == lane, 1.0, 0.0)
        sel_xw = jnp.where(ccol == lane, norm, 0.0)
        w2d = lax.dot_general(sel_y, sel_xw, (((0,), (0,)), ((), ())),
                              preferred_element_type=jnp.float32,
                              precision=lax.Precision.HIGHEST)
        w_ref[0, :, 0, :, :] = jnp.broadcast_to(w2d[None], (w_ref.shape[1], K, K))

    TB = feat_ref.shape[1]
    d = feat_ref.shape[3]
    St_n = st_scr[...]
    for ti in range(0, TB, 2):
        rhs = jnp.concatenate([feat_ref[0, ti], feat_ref[0, ti + 1]], axis=1)
        res = jnp.dot(St_n, rhs,
                      preferred_element_type=jnp.float32,
                      precision=lax.Precision.DEFAULT)
        out_ref[0, ti] = res[:, :d]
        out_ref[0, ti + 1] = res[:, d:]


def kernel(patch_features, coord_map):
    b, t, d, hp, wp = patch_features.shape
    P = hp * wp
    TB = 4
    feats = patch_features.transpose(0, 1, 3, 4, 2).reshape(b, t, P, d)
    coords = coord_map.reshape(b, P, 2).transpose(0, 2, 1)

    grid = (b, t // TB)
    out, weights = pl.pallas_call(
        _splat_body,
        grid=grid,
        in_specs=[
            pl.BlockSpec((1, 2, P), lambda i, j: (i, 0, 0)),
            pl.BlockSpec((1, TB, P, d), lambda i, j: (i, j, 0, 0)),
        ],
        out_specs=[
            pl.BlockSpec((1, TB, KK, d), lambda i, j: (i, j, 0, 0)),
            pl.BlockSpec((1, t, 1, K, K), lambda i, j: (i, 0, 0, 0, 0)),
        ],
        out_shape=[
            jax.ShapeDtypeStruct((b, t, KK, d), jnp.float32),
            jax.ShapeDtypeStruct((b, t, 1, K, K), jnp.float32),
        ],
        scratch_shapes=[pltpu.VMEM((KK, P), jnp.float32)],
    )(coords, feats)
    world = out.reshape(b, t, K, K, d).transpose(0, 1, 4, 2, 3)
    return (world, weights)
